# baseline (speedup 1.0000x reference)
import functools

import jax
import jax.numpy as jnp
from jax import lax
from jax.experimental import pallas as pl
from jax.experimental.pallas import tpu as pltpu
from jax.experimental.pallas import tpu_sc as plsc

_NUM_WORKERS = 32
_CHUNK = 128


def _sc_gather(table, idx):
    B = idx.shape[0]
    D = table.shape[1]
    per_w = B // _NUM_WORKERS
    chunks = per_w // _CHUNK
    idx3 = idx.reshape(_NUM_WORKERS, chunks, _CHUNK)
    mesh = plsc.VectorSubcoreMesh(core_axis_name="c", subcore_axis_name="s")

    @functools.partial(
        pl.kernel,
        mesh=mesh,
        out_type=jax.ShapeDtypeStruct((B, D), jnp.float32),
        scratch_types=[
            pltpu.VMEM((chunks, _CHUNK), jnp.int32),
            pltpu.VMEM((per_w, D), jnp.float32),
            pltpu.SemaphoreType.DMA,
        ],
    )
    def gather_kernel(table_hbm, idx_hbm, out_hbm, idx_v, rows_v, sem):
        wid = lax.axis_index("s") * 2 + lax.axis_index("c")
        pltpu.sync_copy(idx_hbm.at[wid], idx_v)
        copies = []
        for j in range(chunks):
            copies.append(
                pltpu.async_copy(
                    table_hbm.at[idx_v.at[j]],
                    rows_v.at[pl.ds(j * _CHUNK, _CHUNK)],
                    sem,
                )
            )
        for c in copies:
            c.wait()
        pltpu.sync_copy(rows_v, out_hbm.at[pl.ds(wid * per_w, per_w)])

    return gather_kernel(table, idx3)


def _make_tc_body(blk, n_rel, d_pad, m_total):
    inv_m = 1.0 / m_total

    def body(rows_ref, r_ref, rel_ref, w_ref, out_ref):
        i = pl.program_id(0)
        x3 = rows_ref[...]
        x = x3.reshape(3 * blk, x3.shape[2])
        p = jnp.dot(x.astype(jnp.bfloat16), w_ref[...],
                    preferred_element_type=jnp.float32)
        rb = r_ref[...]
        rk3 = jnp.concatenate([rb, rb, rb], axis=0)

        def nrm(v):
            n = jnp.sqrt(jnp.sum(v * v, axis=1, keepdims=True))
            return v / jnp.maximum(n, 1e-12)

        lvl = [p[:, k * d_pad:(k + 1) * d_pad] for k in range(n_rel)]
        rel_n = nrm(rel_ref[...])
        rlvl = [rel_n[k:k + 1, :] for k in range(n_rel)]
        b = 0
        while len(lvl) > 1:
            m3 = (rk3 & (1 << b)) != 0
            mb = (rb & (1 << b)) != 0
            lvl = [jnp.where(m3, lvl[2 * j + 1], lvl[2 * j])
                   for j in range(len(lvl) // 2)]
            rlvl = [jnp.where(mb, rlvl[2 * j + 1], rlvl[2 * j])
                    for j in range(len(rlvl) // 2)]
            b += 1
        sel = lvl[0]
        sel_n = nrm(sel)
        r_vec = rlvl[0]
        h_vec = sel_n[0:blk]
        p_vec = sel_n[blk:2 * blk]
        n_vec = sel_n[2 * blk:3 * blk]
        base = h_vec + r_vec
        pos_s = jnp.sum((base - p_vec) ** 2, axis=1, keepdims=True)
        neg_s = jnp.sum((base - n_vec) ** 2, axis=1, keepdims=True)
        z = pos_s - neg_s
        l_part = jnp.sum(jnp.maximum(z, 0.0) + jnp.log(1.0 + jnp.exp(-jnp.abs(z))))
        reg = 0.5 * (jnp.sum(h_vec * h_vec) + jnp.sum(p_vec * p_vec)
                     + jnp.sum(n_vec * n_vec) + jnp.sum(r_vec * r_vec))
        part = l_part + 0.01 * reg

        @pl.when(i == 0)
        def _init():
            out_ref[...] = jnp.zeros_like(out_ref)

        out_ref[...] += part

        @pl.when(i == pl.num_programs(0) - 1)
        def _finish():
            out_ref[...] = out_ref[...] * inv_m

    return body


def _tc_loss(rows3, r2d, rel_pad, w_pad, m_total, blk=1024):
    m_this = r2d.shape[0]
    d_in = rows3.shape[2]
    n_rel, d_pad = rel_pad.shape
    grid = (m_this // blk,)
    return pl.pallas_call(
        _make_tc_body(blk, n_rel, d_pad, m_total),
        grid=grid,
        in_specs=[
            pl.BlockSpec((3, blk, d_in), lambda i: (0, i, 0)),
            pl.BlockSpec((blk, 1), lambda i: (i, 0)),
            pl.BlockSpec((n_rel, d_pad), lambda i: (0, 0)),
            pl.BlockSpec((d_in, n_rel * d_pad), lambda i: (0, 0)),
        ],
        out_specs=pl.BlockSpec((1, 1), lambda i: (0, 0)),
        out_shape=jax.ShapeDtypeStruct((1, 1), jnp.float32),
    )(rows3, r2d, rel_pad, w_pad)


def kernel(h, r, pos_t, neg_t, entity_embed, relation_embed, W_R):
    m = h.shape[0]
    h = h.astype(jnp.int32)
    r = r.astype(jnp.int32)
    pos_t = pos_t.astype(jnp.int32)
    neg_t = neg_t.astype(jnp.int32)
    n_rel, d_in, d_rel = W_R.shape
    d_pad = 128
    w_padded = jnp.pad(W_R, ((0, 0), (0, 0), (0, d_pad - d_rel)))
    w_all = jnp.transpose(w_padded, (1, 0, 2)).reshape(d_in, n_rel * d_pad)
    w_all = w_all.astype(jnp.bfloat16)
    rel_pad = jnp.pad(relation_embed, ((0, 0), (0, d_pad - d_rel)))

    idx_all = jnp.concatenate([h, pos_t, neg_t], axis=0)
    rows = _sc_gather(entity_embed, idx_all)
    rows3 = rows.reshape(3, m, d_in)
    r2d = r.reshape(m, 1)
    loss2 = _tc_loss(rows3, r2d, rel_pad, w_all, m)
    return loss2[0, 0]


# speedup vs baseline: 1.0367x; 1.0367x over previous
import functools

import jax
import jax.numpy as jnp
from jax import lax
from jax.experimental import pallas as pl
from jax.experimental.pallas import tpu as pltpu
from jax.experimental.pallas import tpu_sc as plsc

_NUM_WORKERS = 32
_CHUNK = 128


def _sc_gather(table, idx):
    B = idx.shape[0]
    D = table.shape[1]
    per_w = B // _NUM_WORKERS
    chunks = per_w // _CHUNK
    idx3 = idx.reshape(_NUM_WORKERS, chunks, _CHUNK)
    mesh = plsc.VectorSubcoreMesh(core_axis_name="c", subcore_axis_name="s")

    @functools.partial(
        pl.kernel,
        mesh=mesh,
        out_type=jax.ShapeDtypeStruct((B, D), jnp.float32),
        scratch_types=[
            pltpu.VMEM((chunks, _CHUNK), jnp.int32),
            pltpu.VMEM((per_w, D), jnp.float32),
            pltpu.SemaphoreType.DMA,
        ],
    )
    def gather_kernel(table_hbm, idx_hbm, out_hbm, idx_v, rows_v, sem):
        wid = lax.axis_index("s") * 2 + lax.axis_index("c")
        pltpu.sync_copy(idx_hbm.at[wid], idx_v)
        copies = []
        for j in range(chunks):
            copies.append(
                pltpu.async_copy(
                    table_hbm.at[idx_v.at[j]],
                    rows_v.at[pl.ds(j * _CHUNK, _CHUNK)],
                    sem,
                )
            )
        for c in copies:
            c.wait()
        pltpu.sync_copy(rows_v, out_hbm.at[pl.ds(wid * per_w, per_w)])

    return gather_kernel(table, idx3)


def _make_tc_body(blk, n_rel, d_pad, m_total):
    inv_m = 1.0 / m_total

    def body(rows_ref, r_ref, rel_ref, w_ref, out_ref):
        i = pl.program_id(0)
        x3 = rows_ref[...]
        x = x3.reshape(3 * blk, x3.shape[2])
        p = jnp.dot(x.astype(jnp.bfloat16), w_ref[...],
                    preferred_element_type=jnp.float32)
        rb = r_ref[...]
        rk3 = jnp.concatenate([rb, rb, rb], axis=0)

        def nrm(v):
            n2 = jnp.sum(v * v, axis=1, keepdims=True)
            return v * lax.rsqrt(jnp.maximum(n2, 1e-24))

        lvl = [p[:, k * d_pad:(k + 1) * d_pad] for k in range(n_rel)]
        rel_n = nrm(rel_ref[...])
        rlvl = [rel_n[k:k + 1, :] for k in range(n_rel)]
        b = 0
        while len(lvl) > 1:
            m3 = (rk3 & (1 << b)) != 0
            mb = (rb & (1 << b)) != 0
            lvl = [jnp.where(m3, lvl[2 * j + 1], lvl[2 * j])
                   for j in range(len(lvl) // 2)]
            rlvl = [jnp.where(mb, rlvl[2 * j + 1], rlvl[2 * j])
                    for j in range(len(rlvl) // 2)]
            b += 1
        sel = lvl[0]
        sel_n = nrm(sel)
        r_vec = rlvl[0]
        h_vec = sel_n[0:blk]
        p_vec = sel_n[blk:2 * blk]
        n_vec = sel_n[2 * blk:3 * blk]
        base = h_vec + r_vec
        pos_s = jnp.sum((base - p_vec) ** 2, axis=1, keepdims=True)
        neg_s = jnp.sum((base - n_vec) ** 2, axis=1, keepdims=True)
        z = pos_s - neg_s
        l_part = jnp.sum(jnp.maximum(z, 0.0) + jnp.log(1.0 + jnp.exp(-jnp.abs(z))))
        reg = 0.5 * (jnp.sum(h_vec * h_vec) + jnp.sum(p_vec * p_vec)
                     + jnp.sum(n_vec * n_vec) + jnp.sum(r_vec * r_vec))
        part = l_part + 0.01 * reg

        @pl.when(i == 0)
        def _init():
            out_ref[...] = jnp.zeros_like(out_ref)

        out_ref[...] += part

        @pl.when(i == pl.num_programs(0) - 1)
        def _finish():
            out_ref[...] = out_ref[...] * inv_m

    return body


def _tc_loss(rows3, r2d, rel_pad, w_pad, m_total, blk=1024):
    m_this = r2d.shape[0]
    d_in = rows3.shape[2]
    n_rel, d_pad = rel_pad.shape
    grid = (m_this // blk,)
    return pl.pallas_call(
        _make_tc_body(blk, n_rel, d_pad, m_total),
        grid=grid,
        in_specs=[
            pl.BlockSpec((3, blk, d_in), lambda i: (0, i, 0)),
            pl.BlockSpec((blk, 1), lambda i: (i, 0)),
            pl.BlockSpec((n_rel, d_pad), lambda i: (0, 0)),
            pl.BlockSpec((d_in, n_rel * d_pad), lambda i: (0, 0)),
        ],
        out_specs=pl.BlockSpec((1, 1), lambda i: (0, 0)),
        out_shape=jax.ShapeDtypeStruct((1, 1), jnp.float32),
    )(rows3, r2d, rel_pad, w_pad)


def kernel(h, r, pos_t, neg_t, entity_embed, relation_embed, W_R):
    m = h.shape[0]
    h = h.astype(jnp.int32)
    r = r.astype(jnp.int32)
    pos_t = pos_t.astype(jnp.int32)
    neg_t = neg_t.astype(jnp.int32)
    n_rel, d_in, d_rel = W_R.shape
    d_pad = 128
    w_padded = jnp.pad(W_R.astype(jnp.bfloat16),
                       ((0, 0), (0, 0), (0, d_pad - d_rel)))
    w_all = jnp.transpose(w_padded, (1, 0, 2)).reshape(d_in, n_rel * d_pad)
    rel_pad = jnp.pad(relation_embed, ((0, 0), (0, d_pad - d_rel)))

    idx_all = jnp.concatenate([h, pos_t, neg_t], axis=0)
    rows = _sc_gather(entity_embed, idx_all)
    rows3 = rows.reshape(3, m, d_in)
    r2d = r.reshape(m, 1)
    loss2 = _tc_loss(rows3, r2d, rel_pad, w_all, m)
    return loss2[0, 0]
